# Initial kernel scaffold; baseline (speedup 1.0000x reference)
import jax
import jax.numpy as jnp
from jax.experimental import pallas as pl


def kernel(x, radial_weights_prev, edge_index, W):
    raise NotImplementedError("write your pallas kernel here")


# speedup vs baseline: 2.0614x; 2.0614x over previous
import functools

import jax
import jax.numpy as jnp
import numpy as np
from jax import lax
from jax.experimental import pallas as pl
from jax.experimental.pallas import tpu as pltpu
from jax.experimental.pallas import tpu_sc as plsc

_N_NODES = 10000
_N_EDGES = 320000
_D_FEAT = 128
_SCALAR_DIM = 64
_R_PREV = 16
_OUT_DIM = _R_PREV + 2 * _SCALAR_DIM

_CHUNK = 128
_N_CHUNKS = _N_EDGES // _CHUNK
_NC = 2
_NS = 16
_NW = _NC * _NS
_CHUNKS_PER_W = (_N_CHUNKS + _NW - 1) // _NW

_INV_SQRT_FAN_IN = np.float32(1.0 / np.sqrt(np.float32(_D_FEAT)))


def _matmul_body(x_ref, w_ref, o_ref):
    o_ref[...] = jax.lax.dot_general(
        x_ref[...], w_ref[...],
        dimension_numbers=(((1,), (0,)), ((), ())),
        preferred_element_type=jnp.float32,
    ) * _INV_SQRT_FAN_IN


_node_linear = pl.pallas_call(
    _matmul_body,
    out_shape=jax.ShapeDtypeStruct((_N_NODES, _SCALAR_DIM), jnp.float32),
)


def _gather_body(feat, esrc, edst, prev, out, isrc_v, idst_v, rsrc_v, rdst_v, prev_v, sem):
    wid = lax.axis_index("s") * _NC + lax.axis_index("c")

    def body(i, carry):
        cid = i * _NW + wid

        @pl.when(cid < _N_CHUNKS)
        def _():
            r0 = pl.multiple_of(cid * _CHUNK, _CHUNK)
            pltpu.sync_copy(esrc.at[pl.ds(r0, _CHUNK)], isrc_v)
            pltpu.sync_copy(edst.at[pl.ds(r0, _CHUNK)], idst_v)
            pltpu.async_copy(feat.at[isrc_v], rsrc_v, sem).wait()
            pltpu.async_copy(feat.at[idst_v], rdst_v, sem).wait()
            pltpu.sync_copy(prev.at[pl.ds(r0, _CHUNK), :], prev_v)
            pltpu.sync_copy(prev_v, out.at[pl.ds(r0, _CHUNK), pl.ds(0, _R_PREV)])
            pltpu.sync_copy(rsrc_v, out.at[pl.ds(r0, _CHUNK), pl.ds(_R_PREV, _SCALAR_DIM)])
            pltpu.sync_copy(rdst_v, out.at[pl.ds(r0, _CHUNK), pl.ds(_R_PREV + _SCALAR_DIM, _SCALAR_DIM)])

        return carry

    lax.fori_loop(0, _CHUNKS_PER_W, body, 0)


_gather_concat = functools.partial(
    pl.kernel,
    out_type=jax.ShapeDtypeStruct((_N_EDGES, _OUT_DIM), jnp.float32),
    mesh=plsc.VectorSubcoreMesh(
        core_axis_name="c", subcore_axis_name="s", num_cores=_NC, num_subcores=_NS
    ),
    scratch_types=[
        pltpu.VMEM((_CHUNK,), jnp.int32),
        pltpu.VMEM((_CHUNK,), jnp.int32),
        pltpu.VMEM((_CHUNK, _SCALAR_DIM), jnp.float32),
        pltpu.VMEM((_CHUNK, _SCALAR_DIM), jnp.float32),
        pltpu.VMEM((_CHUNK, _R_PREV), jnp.float32),
        pltpu.SemaphoreType.DMA,
    ],
    compiler_params=pltpu.CompilerParams(use_tc_tiling_on_sc=False),
)(_gather_body)


@jax.jit
def kernel(x, radial_weights_prev, edge_index, W):
    feat = _node_linear(x, W)
    edge_src = edge_index[1]
    edge_dst = edge_index[0]
    return _gather_concat(feat, edge_src, edge_dst, radial_weights_prev)
